# baseline (speedup 1.0000x reference)
import jax
import jax.numpy as jnp
from jax import lax
from jax.experimental import pallas as pl
from jax.experimental.pallas import tpu as pltpu
from jax.experimental.pallas import tpu_sc as plsc

_WORD_DROPOUT = 0.1
_VOCAB = 100000
_D = 128

_NC = 2
_NS = 16
_NW = _NC * _NS
_L = 16

_NSENT = 4096
_SLEN = 50
_SPAD = 56
_NSPLIT = 4
_NSENT_C = _NSENT // _NSPLIT
_SENT_PER_W = _NSENT_C // _NW
_CHUNK = 2 * _SPAD
_NCHUNK = _SENT_PER_W // 2
_PER_W = _NCHUNK * _CHUNK
_NBUF = 8
_BITS_W = 3200


def _sc_body(table_hbm, idx_hbm, bits_hbm, skeep_hbm, sdrop_hbm, out_hbm,
             idx_v, bits_v, skeep_v, sdrop_v, scales_v, rows_v,
             gsems, ssems):
    wid = lax.axis_index("s") * _NC + lax.axis_index("c")
    sbase = wid * _SENT_PER_W

    pltpu.sync_copy(idx_hbm.at[pl.ds(wid * _PER_W, _PER_W)], idx_v)
    pltpu.sync_copy(bits_hbm, bits_v)
    pltpu.sync_copy(skeep_hbm, skeep_v)
    pltpu.sync_copy(sdrop_hbm, sdrop_v)

    def start_gather(c, slot):
        pltpu.async_copy(table_hbm.at[idx_v.at[pl.ds(c * _CHUNK, _CHUNK)]],
                         rows_v.at[slot], gsems.at[slot])

    def wait_gather(c, slot):
        pltpu.make_async_copy(
            table_hbm.at[idx_v.at[pl.ds(c * _CHUNK, _CHUNK)]],
            rows_v.at[slot], gsems.at[slot]).wait()

    def start_scatter(c, slot):
        pltpu.async_copy(rows_v.at[slot].at[pl.ds(0, _SLEN)],
                         out_hbm.at[sbase + 2 * c], ssems.at[slot])
        pltpu.async_copy(rows_v.at[slot].at[pl.ds(_SPAD, _SLEN)],
                         out_hbm.at[sbase + 2 * c + 1], ssems.at[slot])

    def drain_scatter(c, slot):
        pltpu.make_async_copy(rows_v.at[slot].at[pl.ds(0, _SLEN)],
                              out_hbm.at[sbase + 2 * c],
                              ssems.at[slot]).wait()
        pltpu.make_async_copy(rows_v.at[slot].at[pl.ds(_SPAD, _SLEN)],
                              out_hbm.at[sbase + 2 * c + 1],
                              ssems.at[slot]).wait()

    start_gather(0, 0)
    start_gather(1, 1)
    start_gather(2, 2)
    start_gather(3, 3)

    @pl.loop(0, _NCHUNK // _NBUF)
    def _quad(q):
        cb = q * _NBUF
        for j in range(_NBUF):
            c = cb + j
            nslot = (j + 4) % _NBUF

            @pl.when(c >= 4)
            def _():
                drain_scatter(c - 4, nslot)

            @pl.when(c + 4 < _NCHUNK)
            def _():
                start_gather(c + 4, nslot)

            s_keep = skeep_v[...]
            s_drop = sdrop_v[...]
            for p in range(_CHUNK // _L):
                iv = idx_v[pl.ds(c * _CHUNK + p * _L, _L)]
                w = plsc.load_gather(bits_v, [lax.shift_right_logical(iv, 5)])
                bit = lax.shift_right_logical(w, iv & 31) & 1
                scales_v[pl.ds(p * _L, _L)] = jnp.where(bit == 1, s_keep,
                                                        s_drop)

            wait_gather(c, j)

            @pl.loop(0, _CHUNK, unroll=4)
            def _row(rr):
                sc = plsc.load_gather(scales_v,
                                      [jnp.full((_L,), rr, jnp.int32)])
                for p in range(_D // _L):
                    rows_v[j, rr, pl.ds(p * _L, _L)] = (
                        rows_v[j, rr, pl.ds(p * _L, _L)] * sc)

            start_scatter(c, j)

    for t in range(4):
        drain_scatter(_NCHUNK - 4 + t, (_NCHUNK - 4 + t) % _NBUF)


def kernel(sentences, embedding_matrix, training):
    p = _WORD_DROPOUT
    keep = jax.random.bernoulli(
        jax.random.key(42), 1.0 - p, (embedding_matrix.shape[0], 1))[:, 0]
    keep_pad = jnp.zeros((_BITS_W * 32,), jnp.uint32).at[:_VOCAB].set(
        keep.astype(jnp.uint32))
    bits = (keep_pad.reshape(_BITS_W, 32)
            << jnp.arange(32, dtype=jnp.uint32)[None, :]).sum(
                axis=1, dtype=jnp.uint32).astype(jnp.int32)
    s_drop = jnp.full((_L,), jnp.where(training, 0.0, 1.0), jnp.float32)
    s_keep = jnp.full((_L,), jnp.where(training, 1.0 / (1.0 - p), 1.0),
                      jnp.float32)

    padv = jnp.arange(_NSENT * (_SPAD - _SLEN), dtype=jnp.int32) % 256
    idx = jnp.concatenate(
        [sentences.astype(jnp.int32),
         padv.reshape(_NSENT, _SPAD - _SLEN)], axis=1).reshape(-1)

    mesh = plsc.VectorSubcoreMesh(core_axis_name="c", subcore_axis_name="s")
    call = pl.kernel(
        _sc_body,
        out_type=jax.ShapeDtypeStruct((_NSENT_C, _SLEN, _D), jnp.float32),
        mesh=mesh,
        compiler_params=pltpu.CompilerParams(needs_layout_passes=False),
        scratch_types=[
            pltpu.VMEM((_PER_W,), jnp.int32),
            pltpu.VMEM((_BITS_W,), jnp.int32),
            pltpu.VMEM((_L,), jnp.float32),
            pltpu.VMEM((_L,), jnp.float32),
            pltpu.VMEM((_CHUNK,), jnp.float32),
            pltpu.VMEM((_NBUF, _CHUNK, _D), jnp.float32),
            pltpu.SemaphoreType.DMA((_NBUF,)),
            pltpu.SemaphoreType.DMA((_NBUF,)),
        ],
    )
    per_call = _NSENT_C * _SPAD
    outs = [call(embedding_matrix,
                 lax.dynamic_slice_in_dim(idx, sp * per_call, per_call),
                 bits, s_keep, s_drop)
            for sp in range(_NSPLIT)]
    return jnp.concatenate(outs, axis=0)


# speedup vs baseline: 1.7910x; 1.7910x over previous
import jax
import jax.numpy as jnp
from jax import lax
from jax.experimental import pallas as pl
from jax.experimental.pallas import tpu as pltpu
from jax.experimental.pallas import tpu_sc as plsc

_WORD_DROPOUT = 0.1
_VOCAB = 100000
_D = 128

_NC = 2
_NS = 16
_NW = _NC * _NS
_L = 16

_NSENT = 4096
_SLEN = 50
_SPAD = 56
_SENT_PER_W = _NSENT // _NW
_CHUNK = 2 * _SPAD
_NCHUNK = _SENT_PER_W // 2
_PER_W = _NCHUNK * _CHUNK
_NBUF = 8
_BITS_W = 3200


def _sc_body(table_hbm, idx_hbm, bits_hbm, skeep_hbm, sdrop_hbm, out_hbm,
             idx_v, bits_v, skeep_v, sdrop_v, scales_v, rows_v,
             gsems, ssems):
    wid = lax.axis_index("s") * _NC + lax.axis_index("c")
    sbase = wid * _SENT_PER_W

    pltpu.sync_copy(idx_hbm.at[pl.ds(wid * _PER_W, _PER_W)], idx_v)
    pltpu.sync_copy(bits_hbm, bits_v)
    pltpu.sync_copy(skeep_hbm, skeep_v)
    pltpu.sync_copy(sdrop_hbm, sdrop_v)

    def start_gather(c, slot):
        pltpu.async_copy(table_hbm.at[idx_v.at[pl.ds(c * _CHUNK, _CHUNK)]],
                         rows_v.at[slot], gsems.at[slot])

    def wait_gather(c, slot):
        pltpu.make_async_copy(
            table_hbm.at[idx_v.at[pl.ds(c * _CHUNK, _CHUNK)]],
            rows_v.at[slot], gsems.at[slot]).wait()

    def start_scatter(c, slot):
        pltpu.async_copy(rows_v.at[slot].at[pl.ds(0, _SLEN)],
                         out_hbm.at[sbase + 2 * c], ssems.at[slot])
        pltpu.async_copy(rows_v.at[slot].at[pl.ds(_SPAD, _SLEN)],
                         out_hbm.at[sbase + 2 * c + 1], ssems.at[slot])

    def drain_scatter(c, slot):
        pltpu.make_async_copy(rows_v.at[slot].at[pl.ds(0, _SLEN)],
                              out_hbm.at[sbase + 2 * c],
                              ssems.at[slot]).wait()
        pltpu.make_async_copy(rows_v.at[slot].at[pl.ds(_SPAD, _SLEN)],
                              out_hbm.at[sbase + 2 * c + 1],
                              ssems.at[slot]).wait()

    start_gather(0, 0)
    start_gather(1, 1)
    start_gather(2, 2)
    start_gather(3, 3)

    @pl.loop(0, _NCHUNK // _NBUF)
    def _quad(q):
        cb = q * _NBUF
        for j in range(_NBUF):
            c = cb + j
            nslot = (j + 4) % _NBUF

            @pl.when(c >= 4)
            def _():
                drain_scatter(c - 4, nslot)

            @pl.when(c + 4 < _NCHUNK)
            def _():
                start_gather(c + 4, nslot)

            s_keep = skeep_v[...]
            s_drop = sdrop_v[...]
            for p in range(_CHUNK // _L):
                iv = idx_v[pl.ds(c * _CHUNK + p * _L, _L)]
                w = plsc.load_gather(bits_v, [lax.shift_right_logical(iv, 5)])
                bit = lax.shift_right_logical(w, iv & 31) & 1
                scales_v[pl.ds(p * _L, _L)] = jnp.where(bit == 1, s_keep,
                                                        s_drop)

            wait_gather(c, j)

            @pl.loop(0, _CHUNK, unroll=4)
            def _row(rr):
                sc = plsc.load_gather(scales_v,
                                      [jnp.full((_L,), rr, jnp.int32)])
                for p in range(_D // _L):
                    rows_v[j, rr, pl.ds(p * _L, _L)] = (
                        rows_v[j, rr, pl.ds(p * _L, _L)] * sc)

            start_scatter(c, j)

    for t in range(4):
        drain_scatter(_NCHUNK - 4 + t, (_NCHUNK - 4 + t) % _NBUF)


def kernel(sentences, embedding_matrix, training):
    p = _WORD_DROPOUT
    keep = jax.random.bernoulli(
        jax.random.key(42), 1.0 - p, (embedding_matrix.shape[0], 1))[:, 0]
    keep_pad = jnp.zeros((_BITS_W * 32,), jnp.uint32).at[:_VOCAB].set(
        keep.astype(jnp.uint32))
    bits = (keep_pad.reshape(_BITS_W, 32)
            << jnp.arange(32, dtype=jnp.uint32)[None, :]).sum(
                axis=1, dtype=jnp.uint32).astype(jnp.int32)
    s_drop = jnp.full((_L,), jnp.where(training, 0.0, 1.0), jnp.float32)
    s_keep = jnp.full((_L,), jnp.where(training, 1.0 / (1.0 - p), 1.0),
                      jnp.float32)

    padv = jnp.arange(_NSENT * (_SPAD - _SLEN), dtype=jnp.int32) % 256
    idx = jnp.concatenate(
        [sentences.astype(jnp.int32),
         padv.reshape(_NSENT, _SPAD - _SLEN)], axis=1).reshape(-1)

    mesh = plsc.VectorSubcoreMesh(core_axis_name="c", subcore_axis_name="s")
    call = pl.kernel(
        _sc_body,
        out_type=jax.ShapeDtypeStruct((_NSENT, _SLEN, _D), jnp.float32),
        mesh=mesh,
        compiler_params=pltpu.CompilerParams(needs_layout_passes=False,
                                             use_tc_tiling_on_sc=True),
        scratch_types=[
            pltpu.VMEM((_PER_W,), jnp.int32),
            pltpu.VMEM((_BITS_W,), jnp.int32),
            pltpu.VMEM((_L,), jnp.float32),
            pltpu.VMEM((_L,), jnp.float32),
            pltpu.VMEM((_CHUNK,), jnp.float32),
            pltpu.VMEM((_NBUF, _CHUNK, _D), jnp.float32),
            pltpu.SemaphoreType.DMA((_NBUF,)),
            pltpu.SemaphoreType.DMA((_NBUF,)),
        ],
    )
    return call(embedding_matrix, idx, bits, s_keep, s_drop)


# speedup vs baseline: 1.8594x; 1.0382x over previous
import jax
import jax.numpy as jnp
from jax import lax
from jax.experimental import pallas as pl
from jax.experimental.pallas import tpu as pltpu
from jax.experimental.pallas import tpu_sc as plsc

_WORD_DROPOUT = 0.1
_VOCAB = 100000
_D = 128

_NC = 2
_NS = 16
_NW = _NC * _NS
_L = 16

_NSENT = 4096
_SLEN = 50
_IPAD = 52
_SENT_PER_W = _NSENT // _NW
_CHUNK = 2 * _IPAD
_NCHUNK = _SENT_PER_W // 2
_PER_W = _NCHUNK * _CHUNK
_NBUF = 8
_BITS_W = 3200


def _sc_body(table_hbm, idx_hbm, bits_hbm, skeep_hbm, sdrop_hbm, out_hbm,
             idx_v, bits_v, skeep_v, sdrop_v, scales_v, rows_v,
             gsems, ssems):
    wid = lax.axis_index("s") * _NC + lax.axis_index("c")
    sbase = wid * _SENT_PER_W

    pltpu.sync_copy(idx_hbm.at[pl.ds(wid * _PER_W, _PER_W)], idx_v)
    pltpu.sync_copy(bits_hbm, bits_v)
    pltpu.sync_copy(skeep_hbm, skeep_v)
    pltpu.sync_copy(sdrop_hbm, sdrop_v)

    def start_gather(c, slot):
        pltpu.async_copy(table_hbm.at[idx_v.at[pl.ds(c * _CHUNK, _CHUNK)]],
                         rows_v.at[slot], gsems.at[slot])

    def wait_gather(c, slot):
        pltpu.make_async_copy(
            table_hbm.at[idx_v.at[pl.ds(c * _CHUNK, _CHUNK)]],
            rows_v.at[slot], gsems.at[slot]).wait()

    def start_scatter(c, slot):
        pltpu.async_copy(rows_v.at[slot].at[pl.ds(0, _SLEN)],
                         out_hbm.at[sbase + 2 * c], ssems.at[slot])
        pltpu.async_copy(rows_v.at[slot].at[pl.ds(_IPAD, _SLEN)],
                         out_hbm.at[sbase + 2 * c + 1], ssems.at[slot])

    def drain_scatter(c, slot):
        pltpu.make_async_copy(rows_v.at[slot].at[pl.ds(0, _SLEN)],
                              out_hbm.at[sbase + 2 * c],
                              ssems.at[slot]).wait()
        pltpu.make_async_copy(rows_v.at[slot].at[pl.ds(_IPAD, _SLEN)],
                              out_hbm.at[sbase + 2 * c + 1],
                              ssems.at[slot]).wait()

    start_gather(0, 0)
    start_gather(1, 1)
    start_gather(2, 2)
    start_gather(3, 3)

    @pl.loop(0, _NCHUNK // _NBUF)
    def _quad(q):
        cb = q * _NBUF
        for j in range(_NBUF):
            c = cb + j
            nslot = (j + 4) % _NBUF

            @pl.when(c >= 4)
            def _():
                drain_scatter(c - 4, nslot)

            @pl.when(c + 4 < _NCHUNK)
            def _():
                start_gather(c + 4, nslot)

            s_keep = skeep_v[...]
            s_drop = sdrop_v[...]
            starts = [w * _L for w in range(_CHUNK // _L)] + [_CHUNK - _L]
            for st in starts:
                iv = idx_v[pl.ds(c * _CHUNK + st, _L)]
                w = plsc.load_gather(bits_v, [lax.shift_right_logical(iv, 5)])
                bit = lax.shift_right_logical(w, iv & 31) & 1
                scales_v[pl.ds(st, _L)] = jnp.where(bit == 1, s_keep,
                                                    s_drop)

            wait_gather(c, j)

            @pl.loop(0, _CHUNK, unroll=4)
            def _row(rr):
                sc = plsc.load_gather(scales_v,
                                      [jnp.full((_L,), rr, jnp.int32)])
                for p in range(_D // _L):
                    rows_v[j, rr, pl.ds(p * _L, _L)] = (
                        rows_v[j, rr, pl.ds(p * _L, _L)] * sc)

            start_scatter(c, j)

    for t in range(4):
        drain_scatter(_NCHUNK - 4 + t, (_NCHUNK - 4 + t) % _NBUF)


def kernel(sentences, embedding_matrix, training):
    p = _WORD_DROPOUT
    keep = jax.random.bernoulli(
        jax.random.key(42), 1.0 - p, (embedding_matrix.shape[0], 1))[:, 0]
    keep_pad = jnp.zeros((_BITS_W * 32,), jnp.uint32).at[:_VOCAB].set(
        keep.astype(jnp.uint32))
    bits = (keep_pad.reshape(_BITS_W, 32)
            << jnp.arange(32, dtype=jnp.uint32)[None, :]).sum(
                axis=1, dtype=jnp.uint32).astype(jnp.int32)
    s_drop = jnp.full((_L,), jnp.where(training, 0.0, 1.0), jnp.float32)
    s_keep = jnp.full((_L,), jnp.where(training, 1.0 / (1.0 - p), 1.0),
                      jnp.float32)

    padv = jnp.arange(_NSENT * (_IPAD - _SLEN), dtype=jnp.int32) % 256
    idx = jnp.concatenate(
        [sentences.astype(jnp.int32),
         padv.reshape(_NSENT, _IPAD - _SLEN)], axis=1).reshape(-1)

    mesh = plsc.VectorSubcoreMesh(core_axis_name="c", subcore_axis_name="s")
    call = pl.kernel(
        _sc_body,
        out_type=jax.ShapeDtypeStruct((_NSENT, _SLEN, _D), jnp.float32),
        mesh=mesh,
        compiler_params=pltpu.CompilerParams(needs_layout_passes=False,
                                             use_tc_tiling_on_sc=True),
        scratch_types=[
            pltpu.VMEM((_PER_W,), jnp.int32),
            pltpu.VMEM((_BITS_W,), jnp.int32),
            pltpu.VMEM((_L,), jnp.float32),
            pltpu.VMEM((_L,), jnp.float32),
            pltpu.VMEM((_CHUNK,), jnp.float32),
            pltpu.VMEM((_NBUF, _CHUNK, _D), jnp.float32),
            pltpu.SemaphoreType.DMA((_NBUF,)),
            pltpu.SemaphoreType.DMA((_NBUF,)),
        ],
    )
    return call(embedding_matrix, idx, bits, s_keep, s_drop)


# speedup vs baseline: 1.8721x; 1.0068x over previous
import jax
import jax.numpy as jnp
from jax import lax
from jax.experimental import pallas as pl
from jax.experimental.pallas import tpu as pltpu
from jax.experimental.pallas import tpu_sc as plsc

_WORD_DROPOUT = 0.1
_VOCAB = 100000
_D = 128

_NC = 2
_NS = 16
_NW = _NC * _NS
_L = 16

_NSENT = 4096
_SLEN = 50
_IPAD = 52
_SENT_PER_W = _NSENT // _NW
_CHUNK = 2 * _IPAD
_NCHUNK = _SENT_PER_W // 2
_PER_W = _NCHUNK * _CHUNK
_NBUF = 8
_BITS_W = 3200


def _sc_body(table_hbm, idx_hbm, bits_hbm, skeep_hbm, sdrop_hbm, out_hbm,
             idx_v, bits_v, skeep_v, sdrop_v, scales_v, rows_v,
             gsems, ssems):
    wid = lax.axis_index("s") * _NC + lax.axis_index("c")
    sbase = wid * _SENT_PER_W

    pltpu.sync_copy(idx_hbm.at[pl.ds(wid * _PER_W, _PER_W)], idx_v)
    pltpu.sync_copy(bits_hbm, bits_v)
    pltpu.sync_copy(skeep_hbm, skeep_v)
    pltpu.sync_copy(sdrop_hbm, sdrop_v)

    def start_gather(c, slot):
        pltpu.async_copy(table_hbm.at[idx_v.at[pl.ds(c * _CHUNK, _CHUNK)]],
                         rows_v.at[slot], gsems.at[slot])

    def wait_gather(c, slot):
        pltpu.make_async_copy(
            table_hbm.at[idx_v.at[pl.ds(c * _CHUNK, _CHUNK)]],
            rows_v.at[slot], gsems.at[slot]).wait()

    def start_scatter(c, slot):
        pltpu.async_copy(rows_v.at[slot].at[pl.ds(0, _SLEN)],
                         out_hbm.at[sbase + 2 * c], ssems.at[slot])
        pltpu.async_copy(rows_v.at[slot].at[pl.ds(_IPAD, _SLEN)],
                         out_hbm.at[sbase + 2 * c + 1], ssems.at[slot])

    def drain_scatter(c, slot):
        pltpu.make_async_copy(rows_v.at[slot].at[pl.ds(0, _SLEN)],
                              out_hbm.at[sbase + 2 * c],
                              ssems.at[slot]).wait()
        pltpu.make_async_copy(rows_v.at[slot].at[pl.ds(_IPAD, _SLEN)],
                              out_hbm.at[sbase + 2 * c + 1],
                              ssems.at[slot]).wait()

    for pr in range(6):
        start_gather(pr, pr)

    @pl.loop(0, _NCHUNK // _NBUF)
    def _quad(q):
        cb = q * _NBUF
        for j in range(_NBUF):
            c = cb + j
            nslot = (j + 6) % _NBUF

            @pl.when(c >= 2)
            def _():
                drain_scatter(c - 2, nslot)

            @pl.when(c + 6 < _NCHUNK)
            def _():
                start_gather(c + 6, nslot)

            s_keep = skeep_v[...]
            s_drop = sdrop_v[...]
            starts = [w * _L for w in range(_CHUNK // _L)] + [_CHUNK - _L]
            for st in starts:
                iv = idx_v[pl.ds(c * _CHUNK + st, _L)]
                w = plsc.load_gather(bits_v, [lax.shift_right_logical(iv, 5)])
                bit = lax.shift_right_logical(w, iv & 31) & 1
                scales_v[pl.ds(st, _L)] = jnp.where(bit == 1, s_keep,
                                                    s_drop)

            wait_gather(c, j)

            @pl.loop(0, _CHUNK, unroll=4)
            def _row(rr):
                sc = plsc.load_gather(scales_v,
                                      [jnp.full((_L,), rr, jnp.int32)])
                for p in range(_D // _L):
                    rows_v[j, rr, pl.ds(p * _L, _L)] = (
                        rows_v[j, rr, pl.ds(p * _L, _L)] * sc)

            start_scatter(c, j)

    for t in range(2):
        drain_scatter(_NCHUNK - 2 + t, (_NCHUNK - 2 + t) % _NBUF)


def kernel(sentences, embedding_matrix, training):
    p = _WORD_DROPOUT
    keep = jax.random.bernoulli(
        jax.random.key(42), 1.0 - p, (embedding_matrix.shape[0], 1))[:, 0]
    keep_pad = jnp.zeros((_BITS_W * 32,), jnp.uint32).at[:_VOCAB].set(
        keep.astype(jnp.uint32))
    bits = (keep_pad.reshape(_BITS_W, 32)
            << jnp.arange(32, dtype=jnp.uint32)[None, :]).sum(
                axis=1, dtype=jnp.uint32).astype(jnp.int32)
    s_drop = jnp.full((_L,), jnp.where(training, 0.0, 1.0), jnp.float32)
    s_keep = jnp.full((_L,), jnp.where(training, 1.0 / (1.0 - p), 1.0),
                      jnp.float32)

    padv = jnp.arange(_NSENT * (_IPAD - _SLEN), dtype=jnp.int32) % 256
    idx = jnp.concatenate(
        [sentences.astype(jnp.int32),
         padv.reshape(_NSENT, _IPAD - _SLEN)], axis=1).reshape(-1)

    mesh = plsc.VectorSubcoreMesh(core_axis_name="c", subcore_axis_name="s")
    call = pl.kernel(
        _sc_body,
        out_type=jax.ShapeDtypeStruct((_NSENT, _SLEN, _D), jnp.float32),
        mesh=mesh,
        compiler_params=pltpu.CompilerParams(needs_layout_passes=False,
                                             use_tc_tiling_on_sc=True),
        scratch_types=[
            pltpu.VMEM((_PER_W,), jnp.int32),
            pltpu.VMEM((_BITS_W,), jnp.int32),
            pltpu.VMEM((_L,), jnp.float32),
            pltpu.VMEM((_L,), jnp.float32),
            pltpu.VMEM((_CHUNK,), jnp.float32),
            pltpu.VMEM((_NBUF, _CHUNK, _D), jnp.float32),
            pltpu.SemaphoreType.DMA((_NBUF,)),
            pltpu.SemaphoreType.DMA((_NBUF,)),
        ],
    )
    return call(embedding_matrix, idx, bits, s_keep, s_drop)
